# baseline (speedup 1.0000x reference)
import functools

import jax
import jax.numpy as jnp
from jax import lax
from jax.experimental import pallas as pl
from jax.experimental.pallas import tpu as pltpu
from jax.experimental.pallas import tpu_sc as plsc

B = 16384
D = 64
HID = 256
NW = 32
BPW = B // NW
CH = 128


LAG = 24


def _sc_gather_scs(ct3, dt3, cell_idx, drug_idx):
    mesh = plsc.ScalarSubcoreMesh(axis_name="c", num_cores=2)
    half = B // 2

    @functools.partial(
        pl.kernel,
        mesh=mesh,
        compiler_params=pltpu.CompilerParams(use_tc_tiling_on_sc=True),
        out_type=[
            jax.ShapeDtypeStruct((B // 8, 8, D), jnp.float32),
            jax.ShapeDtypeStruct((B // 8, 8, D), jnp.float32),
        ],
        scratch_types=[
            pltpu.SMEM((CH,), jnp.int32),
            pltpu.SMEM((CH,), jnp.int32),
            pltpu.SemaphoreType.DMA,
            pltpu.SemaphoreType.DMA,
            pltpu.SemaphoreType.DMA,
        ],
    )
    def kscs(ct_hbm, dt_hbm, ci_hbm, di_hbm, c_out, d_out,
             ci_s, di_s, isem, csem, dsem):
        cid = lax.axis_index("c")
        base = cid * half

        for ch in range(half // CH):
            cb = base + ch * CH
            pltpu.async_copy(ci_hbm.at[pl.ds(cb, CH)], ci_s, isem).wait()
            pltpu.async_copy(di_hbm.at[pl.ds(cb, CH)], di_s, isem).wait()

            def fire(i, _, cb=cb):
                o = cb + i
                ci = ci_s[i]
                pltpu.async_copy(ct_hbm.at[ci >> 3, ci & 7],
                                 c_out.at[o >> 3, o & 7], csem)
                di = di_s[i]
                pltpu.async_copy(dt_hbm.at[di >> 3, di & 7],
                                 d_out.at[o >> 3, o & 7], dsem)

                @pl.when(i >= LAG)
                def _():
                    pltpu.make_async_copy(
                        ct_hbm.at[0, 0], c_out.at[0, 0], csem).wait()
                    pltpu.make_async_copy(
                        dt_hbm.at[0, 0], d_out.at[0, 0], dsem).wait()
                return _

            lax.fori_loop(0, CH, fire, None)

            def drain(i, _):
                pltpu.make_async_copy(
                    ct_hbm.at[0, 0], c_out.at[0, 0], csem).wait()
                pltpu.make_async_copy(
                    dt_hbm.at[0, 0], d_out.at[0, 0], dsem).wait()
                return _

            lax.fori_loop(0, LAG, drain, None)

    return kscs(ct3, dt3, cell_idx, drug_idx)


def _sc_gather(ct3, dt3, cell_idx, drug_idx):
    mesh = plsc.VectorSubcoreMesh(core_axis_name="c", subcore_axis_name="s")

    @functools.partial(
        pl.kernel,
        mesh=mesh,
        compiler_params=pltpu.CompilerParams(
            use_tc_tiling_on_sc=True, needs_layout_passes=False),
        out_type=[
            jax.ShapeDtypeStruct((B // 8, 8, D), jnp.float32),
            jax.ShapeDtypeStruct((B // 8, 8, D), jnp.float32),
        ],
        scratch_types=[
            pltpu.VMEM((BPW,), jnp.int32),
            pltpu.VMEM((BPW,), jnp.int32),
            pltpu.VMEM((CH // 8, 8, D), jnp.float32),
            pltpu.VMEM((CH // 8, 8, D), jnp.float32),
            pltpu.SemaphoreType.DMA,
            pltpu.SemaphoreType.DMA,
        ],
    )
    def k(ct_hbm, dt_hbm, ci_hbm, di_hbm, c_out, d_out,
          ci_v, di_v, crows, drows, csem, dsem):
        wid = lax.axis_index("s") * 2 + lax.axis_index("c")
        base = wid * BPW
        pltpu.sync_copy(ci_hbm.at[pl.ds(base, BPW)], ci_v)
        pltpu.sync_copy(di_hbm.at[pl.ds(base, BPW)], di_v)
        lanes = lax.iota(jnp.int32, 16)

        for ch in range(BPW // CH):

            def fire(i, _, ch=ch):
                blk = ch * CH + ((i >> 4) << 4)
                lane = i & 15
                cvec = ci_v[pl.ds(blk, 16)]
                dvec = di_v[pl.ds(blk, 16)]
                ci = jnp.max(jnp.where(lanes == lane, cvec, 0))
                di = jnp.max(jnp.where(lanes == lane, dvec, 0))
                pltpu.async_copy(ct_hbm.at[ci >> 3, ci & 7],
                                 crows.at[i >> 3, i & 7], csem)
                pltpu.async_copy(dt_hbm.at[di >> 3, di & 7],
                                 drows.at[i >> 3, i & 7], dsem)

                @pl.when(i >= LAG)
                def _():
                    pltpu.make_async_copy(
                        ct_hbm.at[0, 0], crows.at[0, 0], csem).wait()
                    pltpu.make_async_copy(
                        dt_hbm.at[0, 0], drows.at[0, 0], dsem).wait()
                return _

            lax.fori_loop(0, CH, fire, None)

            def drain(i, _):
                pltpu.make_async_copy(
                    ct_hbm.at[0, 0], crows.at[0, 0], csem).wait()
                pltpu.make_async_copy(
                    dt_hbm.at[0, 0], drows.at[0, 0], dsem).wait()
                return _

            lax.fori_loop(0, LAG, drain, None)
            obase = wid * (BPW // 8) + ch * (CH // 8)
            pltpu.sync_copy(crows, c_out.at[pl.ds(obase, CH // 8)])
            pltpu.sync_copy(drows, d_out.at[pl.ds(obase, CH // 8)])

    return k(ct3, dt3, cell_idx, drug_idx)


def _gelu(x):
    return 0.5 * x * (1.0 + lax.erf(x * 0.7071067811865476))


BLK = 1024


def _mlp_body(c_ref, d_ref, w1c_ref, w1d_ref, b1_ref,
              w2_ref, b2_ref, w3_ref, b3_ref, o_ref):
    c = c_ref[...]
    d = d_ref[...]
    x1 = (jnp.dot(c, w1c_ref[...], preferred_element_type=jnp.float32)
          + jnp.dot(d, w1d_ref[...], preferred_element_type=jnp.float32)
          + b1_ref[...])
    h = _gelu(x1)
    h = _gelu(jnp.dot(h, w2_ref[...], preferred_element_type=jnp.float32)
              + b2_ref[...])
    o_ref[...] = (jnp.dot(h, w3_ref[...], preferred_element_type=jnp.float32)
                  + b3_ref[...])


def _mlp_tc(c, d, W1c, W1d, b1, W2, b2, W3, b3):
    grid = (B // BLK,)
    return pl.pallas_call(
        _mlp_body,
        grid=grid,
        in_specs=[
            pl.BlockSpec((BLK, D), lambda i: (i, 0)),
            pl.BlockSpec((BLK, D), lambda i: (i, 0)),
            pl.BlockSpec((D, HID), lambda i: (0, 0)),
            pl.BlockSpec((D, HID), lambda i: (0, 0)),
            pl.BlockSpec((1, HID), lambda i: (0, 0)),
            pl.BlockSpec((HID, HID), lambda i: (0, 0)),
            pl.BlockSpec((1, HID), lambda i: (0, 0)),
            pl.BlockSpec((HID, 1), lambda i: (0, 0)),
            pl.BlockSpec((1, 1), lambda i: (0, 0)),
        ],
        out_specs=pl.BlockSpec((BLK, 1), lambda i: (i, 0)),
        out_shape=jax.ShapeDtypeStruct((B, 1), jnp.float32),
    )(c, d, W1c, W1d, b1, W2, b2, W3, b3)


def kernel(cell_idx, drug_idx, cell_table, drug_table, W1, b1, W2, b2, W3, b3):
    ci = cell_idx.astype(jnp.int32)
    di = drug_idx.astype(jnp.int32)
    ct3 = cell_table.reshape(cell_table.shape[0] // 8, 8, D)
    dt3 = drug_table.reshape(drug_table.shape[0] // 8, 8, D)
    c3, d3 = _sc_gather_scs(ct3, dt3, ci, di)
    c = c3.reshape(B, D)
    d = d3.reshape(B, D)
    W1c = W1[:D]
    W1d = W1[D:]
    y = _mlp_tc(c, d, W1c, W1d, b1.reshape(1, HID), W2, b2.reshape(1, HID),
                W3, b3.reshape(1, 1))
    return y.reshape(B)


# speedup vs baseline: 1.5371x; 1.5371x over previous
import functools

import jax
import jax.numpy as jnp
from jax import lax
from jax.experimental import pallas as pl
from jax.experimental.pallas import tpu as pltpu
from jax.experimental.pallas import tpu_sc as plsc

B = 16384
D = 64
HID = 256
NW = 32
BPW = B // NW
CH = 128


LAG = 24


HALF_T = 62500


def _sc_gather(ct3, dtt3, dtb3, cell_idx, drug_idx):
    mesh = plsc.VectorSubcoreMesh(core_axis_name="c", subcore_axis_name="s")

    @functools.partial(
        pl.kernel,
        mesh=mesh,
        compiler_params=pltpu.CompilerParams(
            use_tc_tiling_on_sc=True, needs_layout_passes=False),
        out_type=[
            jax.ShapeDtypeStruct((B // 8, 8, D), jnp.float32),
            jax.ShapeDtypeStruct((B // 8, 8, D), jnp.float32),
        ],
        scratch_types=[
            pltpu.VMEM((BPW,), jnp.int32),
            pltpu.VMEM((BPW,), jnp.int32),
            pltpu.VMEM((CH // 8, 8, D), jnp.float32),
            pltpu.VMEM((CH // 8, 8, D), jnp.float32),
            pltpu.SemaphoreType.DMA,
            pltpu.SemaphoreType.DMA,
        ],
    )
    def k(ct_hbm, dtt_hbm, dtb_hbm, ci_hbm, di_hbm, c_out, d_out,
          ci_v, di_v, crows, drows, csem, dsem):
        wid = lax.axis_index("s") * 2 + lax.axis_index("c")
        base = wid * BPW
        pltpu.sync_copy(ci_hbm.at[pl.ds(base, BPW)], ci_v)
        pltpu.sync_copy(di_hbm.at[pl.ds(base, BPW)], di_v)
        lanes = lax.iota(jnp.int32, 16)

        for ch in range(BPW // CH):

            def fire(i, _, ch=ch):
                blk = ch * CH + ((i >> 4) << 4)
                lane = i & 15
                cvec = ci_v[pl.ds(blk, 16)]
                dvec = di_v[pl.ds(blk, 16)]
                ci = jnp.max(jnp.where(lanes == lane, cvec, 0))
                di = jnp.max(jnp.where(lanes == lane, dvec, 0))
                pltpu.async_copy(ct_hbm.at[ci >> 3, ci & 7],
                                 crows.at[i >> 3, i & 7], csem)
                dt = di >> 3
                dr = di & 7

                @pl.when(dt < HALF_T)
                def _top():
                    pltpu.async_copy(dtt_hbm.at[dt, dr],
                                     drows.at[i >> 3, i & 7], dsem)

                @pl.when(dt >= HALF_T)
                def _bot():
                    pltpu.async_copy(dtb_hbm.at[dt - HALF_T, dr],
                                     drows.at[i >> 3, i & 7], dsem)

                @pl.when(i >= LAG)
                def _():
                    pltpu.make_async_copy(
                        ct_hbm.at[0, 0], crows.at[0, 0], csem).wait()
                    pltpu.make_async_copy(
                        dtt_hbm.at[0, 0], drows.at[0, 0], dsem).wait()
                return _

            lax.fori_loop(0, CH, fire, None)

            def drain(i, _):
                pltpu.make_async_copy(
                    ct_hbm.at[0, 0], crows.at[0, 0], csem).wait()
                pltpu.make_async_copy(
                    dtt_hbm.at[0, 0], drows.at[0, 0], dsem).wait()
                return _

            lax.fori_loop(0, LAG, drain, None)
            obase = wid * (BPW // 8) + ch * (CH // 8)
            pltpu.sync_copy(crows, c_out.at[pl.ds(obase, CH // 8)])
            pltpu.sync_copy(drows, d_out.at[pl.ds(obase, CH // 8)])

    return k(ct3, dtt3, dtb3, cell_idx, drug_idx)


def _gelu(x):
    return 0.5 * x * (1.0 + lax.erf(x * 0.7071067811865476))


BLK = 1024


def _mlp_body(c_ref, d_ref, w1c_ref, w1d_ref, b1_ref,
              w2_ref, b2_ref, w3_ref, b3_ref, o_ref):
    c = c_ref[...]
    d = d_ref[...]
    x1 = (jnp.dot(c, w1c_ref[...], preferred_element_type=jnp.float32)
          + jnp.dot(d, w1d_ref[...], preferred_element_type=jnp.float32)
          + b1_ref[...])
    h = _gelu(x1)
    h = _gelu(jnp.dot(h, w2_ref[...], preferred_element_type=jnp.float32)
              + b2_ref[...])
    o_ref[...] = (jnp.dot(h, w3_ref[...], preferred_element_type=jnp.float32)
                  + b3_ref[...])


def _mlp_tc(c, d, W1c, W1d, b1, W2, b2, W3, b3):
    grid = (B // BLK,)
    return pl.pallas_call(
        _mlp_body,
        grid=grid,
        in_specs=[
            pl.BlockSpec((BLK, D), lambda i: (i, 0)),
            pl.BlockSpec((BLK, D), lambda i: (i, 0)),
            pl.BlockSpec((D, HID), lambda i: (0, 0)),
            pl.BlockSpec((D, HID), lambda i: (0, 0)),
            pl.BlockSpec((1, HID), lambda i: (0, 0)),
            pl.BlockSpec((HID, HID), lambda i: (0, 0)),
            pl.BlockSpec((1, HID), lambda i: (0, 0)),
            pl.BlockSpec((HID, 1), lambda i: (0, 0)),
            pl.BlockSpec((1, 1), lambda i: (0, 0)),
        ],
        out_specs=pl.BlockSpec((BLK, 1), lambda i: (i, 0)),
        out_shape=jax.ShapeDtypeStruct((B, 1), jnp.float32),
    )(c, d, W1c, W1d, b1, W2, b2, W3, b3)


def kernel(cell_idx, drug_idx, cell_table, drug_table, W1, b1, W2, b2, W3, b3):
    ci = cell_idx.astype(jnp.int32)
    di = drug_idx.astype(jnp.int32)
    ct3 = cell_table.reshape(cell_table.shape[0] // 8, 8, D)
    half_rows = HALF_T * 8
    dtt3 = drug_table[:half_rows].reshape(HALF_T, 8, D)
    dtb3 = drug_table[half_rows:].reshape(HALF_T, 8, D)
    c3, d3 = _sc_gather(ct3, dtt3, dtb3, ci, di)
    c = c3.reshape(B, D)
    d = d3.reshape(B, D)
    W1c = W1[:D]
    W1d = W1[D:]
    y = _mlp_tc(c, d, W1c, W1d, b1.reshape(1, HID), W2, b2.reshape(1, HID),
                W3, b3.reshape(1, 1))
    return y.reshape(B)


# speedup vs baseline: 2.8916x; 1.8812x over previous
import functools

import jax
import jax.numpy as jnp
from jax import lax
from jax.experimental import pallas as pl
from jax.experimental.pallas import tpu as pltpu
from jax.experimental.pallas import tpu_sc as plsc

B = 16384
D = 64
HID = 256
NW = 32
BPW = B // NW
CH = 128


LAG = 24


def _sc_gather(ct3, dt3, cell_idx, drug_idx):
    mesh = plsc.VectorSubcoreMesh(core_axis_name="c", subcore_axis_name="s")

    @functools.partial(
        pl.kernel,
        mesh=mesh,
        compiler_params=pltpu.CompilerParams(
            use_tc_tiling_on_sc=True, needs_layout_passes=False),
        out_type=[
            jax.ShapeDtypeStruct((B // 8, 8, D), jnp.float32),
            jax.ShapeDtypeStruct((B // 8, 8, D), jnp.float32),
        ],
        scratch_types=[
            pltpu.VMEM((BPW,), jnp.int32),
            pltpu.VMEM((BPW,), jnp.int32),
            pltpu.VMEM((CH // 8, 8, D), jnp.float32),
            pltpu.VMEM((CH // 8, 8, D), jnp.float32),
            pltpu.SemaphoreType.DMA,
            pltpu.SemaphoreType.DMA,
        ],
    )
    def k(ct_hbm, dt_hbm, ci_hbm, di_hbm, c_out, d_out,
          ci_v, di_v, crows, drows, csem, dsem):
        wid = lax.axis_index("s") * 2 + lax.axis_index("c")
        base = wid * BPW
        pltpu.sync_copy(ci_hbm.at[pl.ds(base, BPW)], ci_v)
        pltpu.sync_copy(di_hbm.at[pl.ds(base, BPW)], di_v)
        lanes = lax.iota(jnp.int32, 16)

        for ch in range(BPW // CH):

            def fire(i, _, ch=ch):
                blk = ch * CH + ((i >> 4) << 4)
                lane = i & 15
                cvec = ci_v[pl.ds(blk, 16)]
                dvec = di_v[pl.ds(blk, 16)]
                ci = jnp.max(jnp.where(lanes == lane, cvec, 0))
                di = jnp.max(jnp.where(lanes == lane, dvec, 0))
                pltpu.async_copy(ct_hbm.at[ci >> 3, ci & 7],
                                 crows.at[i >> 3, i & 7], csem)
                pltpu.async_copy(dt_hbm.at[di >> 3, di & 7],
                                 drows.at[i >> 3, i & 7], dsem)

                @pl.when(i >= LAG)
                def _():
                    pltpu.make_async_copy(
                        ct_hbm.at[0, 0], crows.at[0, 0], csem).wait()
                    pltpu.make_async_copy(
                        dt_hbm.at[0, 0], drows.at[0, 0], dsem).wait()
                return _

            lax.fori_loop(0, CH, fire, None)

            def drain(i, _):
                pltpu.make_async_copy(
                    ct_hbm.at[0, 0], crows.at[0, 0], csem).wait()
                pltpu.make_async_copy(
                    dt_hbm.at[0, 0], drows.at[0, 0], dsem).wait()
                return _

            lax.fori_loop(0, LAG, drain, None)
            obase = wid * (BPW // 8) + ch * (CH // 8)
            pltpu.sync_copy(crows, c_out.at[pl.ds(obase, CH // 8)])
            pltpu.sync_copy(drows, d_out.at[pl.ds(obase, CH // 8)])

    return k(ct3, dt3, cell_idx, drug_idx)


def _gelu(x):
    return 0.5 * x * (1.0 + lax.erf(x * 0.7071067811865476))


BLK = 1024


def _mlp_body(c_ref, d_ref, w1c_ref, w1d_ref, b1_ref,
              w2_ref, b2_ref, w3_ref, b3_ref, o_ref):
    c = c_ref[...]
    d = d_ref[...]
    x1 = (jnp.dot(c, w1c_ref[...], preferred_element_type=jnp.float32)
          + jnp.dot(d, w1d_ref[...], preferred_element_type=jnp.float32)
          + b1_ref[...])
    h = _gelu(x1)
    h = _gelu(jnp.dot(h, w2_ref[...], preferred_element_type=jnp.float32)
              + b2_ref[...])
    o_ref[...] = (jnp.dot(h, w3_ref[...], preferred_element_type=jnp.float32)
                  + b3_ref[...])


def _mlp_tc(c, d, W1c, W1d, b1, W2, b2, W3, b3):
    grid = (B // BLK,)
    return pl.pallas_call(
        _mlp_body,
        grid=grid,
        in_specs=[
            pl.BlockSpec((BLK, D), lambda i: (i, 0)),
            pl.BlockSpec((BLK, D), lambda i: (i, 0)),
            pl.BlockSpec((D, HID), lambda i: (0, 0)),
            pl.BlockSpec((D, HID), lambda i: (0, 0)),
            pl.BlockSpec((1, HID), lambda i: (0, 0)),
            pl.BlockSpec((HID, HID), lambda i: (0, 0)),
            pl.BlockSpec((1, HID), lambda i: (0, 0)),
            pl.BlockSpec((HID, 1), lambda i: (0, 0)),
            pl.BlockSpec((1, 1), lambda i: (0, 0)),
        ],
        out_specs=pl.BlockSpec((BLK, 1), lambda i: (i, 0)),
        out_shape=jax.ShapeDtypeStruct((B, 1), jnp.float32),
    )(c, d, W1c, W1d, b1, W2, b2, W3, b3)


def kernel(cell_idx, drug_idx, cell_table, drug_table, W1, b1, W2, b2, W3, b3):
    ci = cell_idx.astype(jnp.int32)
    di = drug_idx.astype(jnp.int32)
    ct3 = cell_table.reshape(cell_table.shape[0] // 8, 8, D)
    dt3 = drug_table.reshape(drug_table.shape[0] // 8, 8, D)
    c3, d3 = _sc_gather(ct3, dt3, ci, di)
    c = c3.reshape(B, D)
    d = d3.reshape(B, D)
    W1c = W1[:D]
    W1d = W1[D:]
    y = _mlp_tc(c, d, W1c, W1d, b1.reshape(1, HID), W2, b2.reshape(1, HID),
                W3, b3.reshape(1, 1))
    return y.reshape(B)


# speedup vs baseline: 2.9640x; 1.0250x over previous
import functools

import jax
import jax.numpy as jnp
from jax import lax
from jax.experimental import pallas as pl
from jax.experimental.pallas import tpu as pltpu
from jax.experimental.pallas import tpu_sc as plsc

B = 16384
D = 64
HID = 256
NW = 32
BPW = B // NW
CH = 128


LAG = 24


def _sc_gather(ct3, dt3, cell_idx, drug_idx, nb=B):
    mesh = plsc.VectorSubcoreMesh(core_axis_name="c", subcore_axis_name="s")
    bpw = nb // NW

    @functools.partial(
        pl.kernel,
        mesh=mesh,
        compiler_params=pltpu.CompilerParams(
            use_tc_tiling_on_sc=True, needs_layout_passes=False),
        out_type=[
            jax.ShapeDtypeStruct((nb // 8, 8, D), jnp.float32),
            jax.ShapeDtypeStruct((nb // 8, 8, D), jnp.float32),
        ],
        scratch_types=[
            pltpu.VMEM((bpw,), jnp.int32),
            pltpu.VMEM((bpw,), jnp.int32),
            pltpu.VMEM((CH // 8, 8, D), jnp.float32),
            pltpu.VMEM((CH // 8, 8, D), jnp.float32),
            pltpu.SemaphoreType.DMA,
            pltpu.SemaphoreType.DMA,
        ],
    )
    def k(ct_hbm, dt_hbm, ci_hbm, di_hbm, c_out, d_out,
          ci_v, di_v, crows, drows, csem, dsem):
        wid = lax.axis_index("s") * 2 + lax.axis_index("c")
        base = wid * bpw
        pltpu.sync_copy(ci_hbm.at[pl.ds(base, bpw)], ci_v)
        pltpu.sync_copy(di_hbm.at[pl.ds(base, bpw)], di_v)
        lanes = lax.iota(jnp.int32, 16)

        for ch in range(bpw // CH):

            def fire(i, _, ch=ch):
                blk = ch * CH + ((i >> 4) << 4)
                lane = i & 15
                cvec = ci_v[pl.ds(blk, 16)]
                dvec = di_v[pl.ds(blk, 16)]
                ci = jnp.max(jnp.where(lanes == lane, cvec, 0))
                di = jnp.max(jnp.where(lanes == lane, dvec, 0))
                pltpu.async_copy(ct_hbm.at[ci >> 3, ci & 7],
                                 crows.at[i >> 3, i & 7], csem)
                pltpu.async_copy(dt_hbm.at[di >> 3, di & 7],
                                 drows.at[i >> 3, i & 7], dsem)

                @pl.when(i >= LAG)
                def _():
                    pltpu.make_async_copy(
                        ct_hbm.at[0, 0], crows.at[0, 0], csem).wait()
                    pltpu.make_async_copy(
                        dt_hbm.at[0, 0], drows.at[0, 0], dsem).wait()
                return _

            lax.fori_loop(0, CH, fire, None)

            def drain(i, _):
                pltpu.make_async_copy(
                    ct_hbm.at[0, 0], crows.at[0, 0], csem).wait()
                pltpu.make_async_copy(
                    dt_hbm.at[0, 0], drows.at[0, 0], dsem).wait()
                return _

            lax.fori_loop(0, LAG, drain, None)
            obase = wid * (bpw // 8) + ch * (CH // 8)
            pltpu.sync_copy(crows, c_out.at[pl.ds(obase, CH // 8)])
            pltpu.sync_copy(drows, d_out.at[pl.ds(obase, CH // 8)])

    return k(ct3, dt3, cell_idx, drug_idx)


def _gelu(x):
    return 0.5 * x * (1.0 + lax.erf(x * 0.7071067811865476))


BLK = 1024


def _mlp_body(c_ref, d_ref, w1c_ref, w1d_ref, b1_ref,
              w2_ref, b2_ref, w3_ref, b3_ref, o_ref):
    c = c_ref[...]
    d = d_ref[...]
    x1 = (jnp.dot(c, w1c_ref[...], preferred_element_type=jnp.float32)
          + jnp.dot(d, w1d_ref[...], preferred_element_type=jnp.float32)
          + b1_ref[...])
    h = _gelu(x1)
    h = _gelu(jnp.dot(h, w2_ref[...], preferred_element_type=jnp.float32)
              + b2_ref[...])
    o_ref[...] = (jnp.dot(h, w3_ref[...], preferred_element_type=jnp.float32)
                  + b3_ref[...])


def _mlp_tc(c, d, W1c, W1d, b1, W2, b2, W3, b3, nb=B):
    grid = (nb // BLK,)
    return pl.pallas_call(
        _mlp_body,
        grid=grid,
        in_specs=[
            pl.BlockSpec((BLK, D), lambda i: (i, 0)),
            pl.BlockSpec((BLK, D), lambda i: (i, 0)),
            pl.BlockSpec((D, HID), lambda i: (0, 0)),
            pl.BlockSpec((D, HID), lambda i: (0, 0)),
            pl.BlockSpec((1, HID), lambda i: (0, 0)),
            pl.BlockSpec((HID, HID), lambda i: (0, 0)),
            pl.BlockSpec((1, HID), lambda i: (0, 0)),
            pl.BlockSpec((HID, 1), lambda i: (0, 0)),
            pl.BlockSpec((1, 1), lambda i: (0, 0)),
        ],
        out_specs=pl.BlockSpec((BLK, 1), lambda i: (i, 0)),
        out_shape=jax.ShapeDtypeStruct((nb, 1), jnp.float32),
    )(c, d, W1c, W1d, b1, W2, b2, W3, b3)


NSPLIT = 2


def kernel(cell_idx, drug_idx, cell_table, drug_table, W1, b1, W2, b2, W3, b3):
    ci = cell_idx.astype(jnp.int32)
    di = drug_idx.astype(jnp.int32)
    ct3 = cell_table.reshape(cell_table.shape[0] // 8, 8, D)
    dt3 = drug_table.reshape(drug_table.shape[0] // 8, 8, D)
    W1c = W1[:D]
    W1d = W1[D:]
    b1r = b1.reshape(1, HID)
    b2r = b2.reshape(1, HID)
    b3r = b3.reshape(1, 1)
    nb = B // NSPLIT
    ys = []
    for h in range(NSPLIT):
        s = slice(h * nb, (h + 1) * nb)
        c3, d3 = _sc_gather(ct3, dt3, ci[s], di[s], nb=nb)
        ys.append(_mlp_tc(c3.reshape(nb, D), d3.reshape(nb, D),
                          W1c, W1d, b1r, W2, b2r, W3, b3r, nb=nb))
    return jnp.concatenate(ys, axis=0).reshape(B)
